# initial kernel scaffold (unmeasured)
import jax
import jax.numpy as jnp
from jax import lax
from jax.experimental import pallas as pl
from jax.experimental.pallas import tpu as pltpu

MH = 4096
D = 4096
CHUNK = 512
NCH = MH // CHUNK
EPS = 1e-6


def kernel(partial, gamma):
    gamma2d = gamma.reshape(1, D)

    def body(p_ref, g_ref, out_ref, recv_ref, a, b, o, local_sems,
             send_sem, recv_sem):
        my_x = lax.axis_index("x")
        my_y = lax.axis_index("y")
        my_z = lax.axis_index("z")
        y_peer = (my_x, 1 - my_y, my_z)

        barrier = pltpu.get_barrier_semaphore()
        pl.semaphore_signal(barrier, inc=1, device_id=y_peer,
                            device_id_type=pl.DeviceIdType.MESH)
        pl.semaphore_wait(barrier, 1)

        rdma = pltpu.make_async_remote_copy(
            src_ref=p_ref.at[0, pl.ds((1 - my_y) * MH, MH), :],
            dst_ref=recv_ref,
            send_sem=send_sem,
            recv_sem=recv_sem,
            device_id=y_peer,
            device_id_type=pl.DeviceIdType.MESH,
        )
        rdma.start()
        rdma.wait()

        for c in range(NCH):
            r0 = c * CHUNK
            cp_a = pltpu.make_async_copy(
                p_ref.at[0, pl.ds(my_y * MH + r0, CHUNK), :], a,
                local_sems.at[0])
            cp_b = pltpu.make_async_copy(
                recv_ref.at[pl.ds(r0, CHUNK), :], b, local_sems.at[1])
            cp_a.start()
            cp_b.start()
            cp_a.wait()
            cp_b.wait()
            y = a[...] + b[...]
            ms = jnp.mean(y * y, axis=-1, keepdims=True)
            o[...] = y * lax.rsqrt(ms + EPS) * g_ref[...]
            cp_o = pltpu.make_async_copy(
                o, out_ref.at[pl.ds(r0, CHUNK), :], local_sems.at[2])
            cp_o.start()
            cp_o.wait()

    return pl.pallas_call(
        body,
        out_shape=jax.ShapeDtypeStruct((MH, D), jnp.float32),
        in_specs=[
            pl.BlockSpec(memory_space=pl.ANY),
            pl.BlockSpec(memory_space=pltpu.MemorySpace.VMEM),
        ],
        out_specs=pl.BlockSpec(memory_space=pl.ANY),
        scratch_shapes=[
            pltpu.MemorySpace.HBM((MH, D), jnp.float32),
            pltpu.VMEM((CHUNK, D), jnp.float32),
            pltpu.VMEM((CHUNK, D), jnp.float32),
            pltpu.VMEM((CHUNK, D), jnp.float32),
            pltpu.SemaphoreType.DMA((3,)),
            pltpu.SemaphoreType.DMA,
            pltpu.SemaphoreType.DMA,
        ],
        compiler_params=pltpu.CompilerParams(collective_id=0),
    )(partial, gamma2d)


# baseline (device time: 871549 ns/iter reference)
import jax
import jax.numpy as jnp
from jax import lax
from jax.experimental import pallas as pl
from jax.experimental.pallas import tpu as pltpu

MH = 4096
D = 4096
CHUNK = 256
NCH = MH // CHUNK
EPS = 1e-6


def kernel(partial, gamma):
    gamma2d = gamma.reshape(1, D)

    def body(p_ref, g_ref, out_ref, recv_ref, a, b, o, local_sems,
             send_sem, recv_sem):
        my_x = lax.axis_index("x")
        my_y = lax.axis_index("y")
        my_z = lax.axis_index("z")
        y_peer = (my_x, 1 - my_y, my_z)

        barrier = pltpu.get_barrier_semaphore()
        pl.semaphore_signal(barrier, inc=1, device_id=y_peer,
                            device_id_type=pl.DeviceIdType.MESH)
        pl.semaphore_wait(barrier, 1)

        rdma = pltpu.make_async_remote_copy(
            src_ref=p_ref.at[0, pl.ds((1 - my_y) * MH, MH), :],
            dst_ref=recv_ref,
            send_sem=send_sem,
            recv_sem=recv_sem,
            device_id=y_peer,
            device_id_type=pl.DeviceIdType.MESH,
        )
        rdma.start()
        rdma.wait()

        for c in range(NCH):
            r0 = c * CHUNK
            cp_a = pltpu.make_async_copy(
                p_ref.at[0, pl.ds(my_y * MH + r0, CHUNK), :], a,
                local_sems.at[0])
            cp_b = pltpu.make_async_copy(
                recv_ref.at[pl.ds(r0, CHUNK), :], b, local_sems.at[1])
            cp_a.start()
            cp_b.start()
            cp_a.wait()
            cp_b.wait()
            y = a[...] + b[...]
            ms = jnp.mean(y * y, axis=-1, keepdims=True)
            o[...] = y * lax.rsqrt(ms + EPS) * g_ref[...]
            cp_o = pltpu.make_async_copy(
                o, out_ref.at[pl.ds(r0, CHUNK), :], local_sems.at[2])
            cp_o.start()
            cp_o.wait()

    out, _ = pl.pallas_call(
        body,
        out_shape=(
            jax.ShapeDtypeStruct((MH, D), jnp.float32),
            jax.ShapeDtypeStruct((MH, D), jnp.float32),
        ),
        in_specs=[
            pl.BlockSpec(memory_space=pl.ANY),
            pl.BlockSpec(memory_space=pltpu.MemorySpace.VMEM),
        ],
        out_specs=(
            pl.BlockSpec(memory_space=pl.ANY),
            pl.BlockSpec(memory_space=pl.ANY),
        ),
        scratch_shapes=[
            pltpu.VMEM((CHUNK, D), jnp.float32),
            pltpu.VMEM((CHUNK, D), jnp.float32),
            pltpu.VMEM((CHUNK, D), jnp.float32),
            pltpu.SemaphoreType.DMA((3,)),
            pltpu.SemaphoreType.DMA,
            pltpu.SemaphoreType.DMA,
        ],
        compiler_params=pltpu.CompilerParams(collective_id=0),
    )(partial, gamma2d)
    return out


# device time: 461778 ns/iter; 1.8874x vs baseline; 1.8874x over previous
import jax
import jax.numpy as jnp
from jax import lax
from jax.experimental import pallas as pl
from jax.experimental.pallas import tpu as pltpu

MH = 4096
D = 4096
HALF = MH // 2
CHUNK = 256
NCH = HALF // CHUNK
EPS = 1e-6


def kernel(partial, gamma):
    gamma2d = gamma.reshape(1, D)

    def body(p_ref, g_ref, out_ref, recv_y_ref, a, b, o, local_sems,
             y_send_sems, y_recv_sems, x_send_sems, x_recv_sems):
        my_x = lax.axis_index("x")
        my_y = lax.axis_index("y")
        my_z = lax.axis_index("z")
        y_peer = (my_x, 1 - my_y, my_z)
        x_peer = (1 - my_x, my_y, my_z)

        barrier = pltpu.get_barrier_semaphore()
        for peer in (y_peer, x_peer):
            pl.semaphore_signal(barrier, inc=1, device_id=peer,
                                device_id_type=pl.DeviceIdType.MESH)
        pl.semaphore_wait(barrier, 2)

        y_rdmas = []
        for c in range(NCH):
            r = my_x * HALF + c * CHUNK
            rdma = pltpu.make_async_remote_copy(
                src_ref=p_ref.at[0, pl.ds((1 - my_y) * MH + r, CHUNK), :],
                dst_ref=recv_y_ref.at[pl.ds(c * CHUNK, CHUNK), :],
                send_sem=y_send_sems.at[c],
                recv_sem=y_recv_sems.at[c],
                device_id=y_peer,
                device_id_type=pl.DeviceIdType.MESH,
            )
            rdma.start()
            y_rdmas.append(rdma)

        x_in = []
        for c in range(NCH):
            x_in.append(pltpu.make_async_remote_copy(
                src_ref=o.at[0],
                dst_ref=out_ref.at[pl.ds((1 - my_x) * HALF + c * CHUNK, CHUNK), :],
                send_sem=x_send_sems.at[c],
                recv_sem=x_recv_sems.at[c],
                device_id=x_peer,
                device_id_type=pl.DeviceIdType.MESH,
            ))

        x_out = []
        o_stores = []
        for c in range(NCH):
            r = my_x * HALF + c * CHUNK
            y_rdmas[c].wait_recv()
            cp_a = pltpu.make_async_copy(
                p_ref.at[0, pl.ds(my_y * MH + r, CHUNK), :], a,
                local_sems.at[0])
            cp_b = pltpu.make_async_copy(
                recv_y_ref.at[pl.ds(c * CHUNK, CHUNK), :], b,
                local_sems.at[1])
            cp_a.start()
            cp_b.start()
            if c >= 2:
                x_out[c - 2].wait_send()
                o_stores[c - 2].wait()
            cp_a.wait()
            cp_b.wait()
            y = a[...] + b[...]
            ms = jnp.mean(y * y, axis=-1, keepdims=True)
            o[c % 2] = y * lax.rsqrt(ms + EPS) * g_ref[...]
            rdma_x = pltpu.make_async_remote_copy(
                src_ref=o.at[c % 2],
                dst_ref=out_ref.at[pl.ds(r, CHUNK), :],
                send_sem=x_send_sems.at[c],
                recv_sem=x_recv_sems.at[c],
                device_id=x_peer,
                device_id_type=pl.DeviceIdType.MESH,
            )
            rdma_x.start()
            x_out.append(rdma_x)
            cp_o = pltpu.make_async_copy(
                o.at[c % 2], out_ref.at[pl.ds(r, CHUNK), :],
                local_sems.at[2 + (c % 2)])
            cp_o.start()
            o_stores.append(cp_o)

        for c in range(NCH):
            x_in[c].wait_recv()
            y_rdmas[c].wait_send()
        for c in range(max(NCH - 2, 0), NCH):
            x_out[c].wait_send()
            o_stores[c].wait()

    out, _ = pl.pallas_call(
        body,
        out_shape=(
            jax.ShapeDtypeStruct((MH, D), jnp.float32),
            jax.ShapeDtypeStruct((HALF, D), jnp.float32),
        ),
        in_specs=[
            pl.BlockSpec(memory_space=pl.ANY),
            pl.BlockSpec(memory_space=pltpu.MemorySpace.VMEM),
        ],
        out_specs=(
            pl.BlockSpec(memory_space=pl.ANY),
            pl.BlockSpec(memory_space=pl.ANY),
        ),
        scratch_shapes=[
            pltpu.VMEM((CHUNK, D), jnp.float32),
            pltpu.VMEM((CHUNK, D), jnp.float32),
            pltpu.VMEM((2, CHUNK, D), jnp.float32),
            pltpu.SemaphoreType.DMA((4,)),
            pltpu.SemaphoreType.DMA((NCH,)),
            pltpu.SemaphoreType.DMA((NCH,)),
            pltpu.SemaphoreType.DMA((NCH,)),
            pltpu.SemaphoreType.DMA((NCH,)),
        ],
        compiler_params=pltpu.CompilerParams(collective_id=0),
    )(partial, gamma2d)
    return out


# device time: 437394 ns/iter; 1.9926x vs baseline; 1.0557x over previous
import jax
import jax.numpy as jnp
from jax import lax
from jax.experimental import pallas as pl
from jax.experimental.pallas import tpu as pltpu

MH = 4096
D = 4096
HALF = MH // 2
CHUNK = 128
NCH = HALF // CHUNK
EPS = 1e-6


def kernel(partial, gamma):
    gamma2d = gamma.reshape(1, D)

    def body(p_ref, g_ref, out_ref, recv_y_ref, a, b, o, local_sems,
             y_send_sems, y_recv_sems, x_send_sems, x_recv_sems):
        my_x = lax.axis_index("x")
        my_y = lax.axis_index("y")
        my_z = lax.axis_index("z")
        y_peer = (my_x, 1 - my_y, my_z)
        x_peer = (1 - my_x, my_y, my_z)

        barrier = pltpu.get_barrier_semaphore()
        for peer in (y_peer, x_peer):
            pl.semaphore_signal(barrier, inc=1, device_id=peer,
                                device_id_type=pl.DeviceIdType.MESH)
        pl.semaphore_wait(barrier, 2)

        y_rdmas = []
        for c in range(NCH):
            r = my_x * HALF + c * CHUNK
            rdma = pltpu.make_async_remote_copy(
                src_ref=p_ref.at[0, pl.ds((1 - my_y) * MH + r, CHUNK), :],
                dst_ref=recv_y_ref.at[pl.ds(c * CHUNK, CHUNK), :],
                send_sem=y_send_sems.at[c],
                recv_sem=y_recv_sems.at[c],
                device_id=y_peer,
                device_id_type=pl.DeviceIdType.MESH,
            )
            rdma.start()
            y_rdmas.append(rdma)

        x_in = []
        for c in range(NCH):
            x_in.append(pltpu.make_async_remote_copy(
                src_ref=o.at[0],
                dst_ref=out_ref.at[pl.ds((1 - my_x) * HALF + c * CHUNK, CHUNK), :],
                send_sem=x_send_sems.at[c],
                recv_sem=x_recv_sems.at[c],
                device_id=x_peer,
                device_id_type=pl.DeviceIdType.MESH,
            ))

        def a_load(c):
            cp = pltpu.make_async_copy(
                p_ref.at[0, pl.ds(my_y * MH + my_x * HALF + c * CHUNK, CHUNK), :],
                a.at[c % 2], local_sems.at[c % 2])
            cp.start()
            return cp

        a_loads = {0: a_load(0)}
        x_out = []
        o_stores = []
        for c in range(NCH):
            r = my_x * HALF + c * CHUNK
            if c + 1 < NCH:
                a_loads[c + 1] = a_load(c + 1)
            y_rdmas[c].wait_recv()
            cp_b = pltpu.make_async_copy(
                recv_y_ref.at[pl.ds(c * CHUNK, CHUNK), :], b,
                local_sems.at[2])
            cp_b.start()
            if c >= 2:
                x_out[c - 2].wait_send()
                o_stores[c - 2].wait()
            a_loads[c].wait()
            cp_b.wait()
            y = a[c % 2] + b[...]
            ms = jnp.mean(y * y, axis=-1, keepdims=True)
            o[c % 2] = y * lax.rsqrt(ms + EPS) * g_ref[...]
            rdma_x = pltpu.make_async_remote_copy(
                src_ref=o.at[c % 2],
                dst_ref=out_ref.at[pl.ds(r, CHUNK), :],
                send_sem=x_send_sems.at[c],
                recv_sem=x_recv_sems.at[c],
                device_id=x_peer,
                device_id_type=pl.DeviceIdType.MESH,
            )
            rdma_x.start()
            x_out.append(rdma_x)
            cp_o = pltpu.make_async_copy(
                o.at[c % 2], out_ref.at[pl.ds(r, CHUNK), :],
                local_sems.at[3 + (c % 2)])
            cp_o.start()
            o_stores.append(cp_o)

        for c in range(NCH):
            x_in[c].wait_recv()
            y_rdmas[c].wait_send()
        for c in range(max(NCH - 2, 0), NCH):
            x_out[c].wait_send()
            o_stores[c].wait()

    out, _ = pl.pallas_call(
        body,
        out_shape=(
            jax.ShapeDtypeStruct((MH, D), jnp.float32),
            jax.ShapeDtypeStruct((HALF, D), jnp.float32),
        ),
        in_specs=[
            pl.BlockSpec(memory_space=pl.ANY),
            pl.BlockSpec(memory_space=pltpu.MemorySpace.VMEM),
        ],
        out_specs=(
            pl.BlockSpec(memory_space=pl.ANY),
            pl.BlockSpec(memory_space=pl.ANY),
        ),
        scratch_shapes=[
            pltpu.VMEM((2, CHUNK, D), jnp.float32),
            pltpu.VMEM((CHUNK, D), jnp.float32),
            pltpu.VMEM((2, CHUNK, D), jnp.float32),
            pltpu.SemaphoreType.DMA((5,)),
            pltpu.SemaphoreType.DMA((NCH,)),
            pltpu.SemaphoreType.DMA((NCH,)),
            pltpu.SemaphoreType.DMA((NCH,)),
            pltpu.SemaphoreType.DMA((NCH,)),
        ],
        compiler_params=pltpu.CompilerParams(collective_id=0),
    )(partial, gamma2d)
    return out
